# baseline (speedup 1.0000x reference)
import functools

import jax
import jax.numpy as jnp
from jax import lax
from jax.experimental import pallas as pl
from jax.experimental.pallas import tpu as pltpu
from jax.experimental.pallas import tpu_sc as plsc

_NC = 2
_NS = 16
_NW = _NC * _NS
_CHUNK = 128


def _sc_gather_body(cid_hbm, did_hbm, words_hbm, dweights_hbm,
                    wv_out, dw_out,
                    cidx_v, didx_v, wrows_v, drows_v, sem):
    nchunk = cidx_v.shape[0]
    wid = lax.axis_index("s") * _NC + lax.axis_index("c")
    base = wid * nchunk
    pltpu.sync_copy(cid_hbm.at[pl.ds(base, nchunk)], cidx_v)
    pltpu.sync_copy(did_hbm.at[pl.ds(base, nchunk)], didx_v)
    copies = []
    for j in range(nchunk):
        copies.append(pltpu.async_copy(words_hbm.at[cidx_v.at[j]],
                                       wrows_v.at[j], sem))
        copies.append(pltpu.async_copy(dweights_hbm.at[didx_v.at[j]],
                                       drows_v.at[j], sem))
    for c in copies:
        c.wait()
    pltpu.sync_copy(wrows_v, wv_out.at[pl.ds(base, nchunk)])
    pltpu.sync_copy(drows_v, dw_out.at[pl.ds(base, nchunk)])


def _make_sc_gather(b, emb, topics, nchunk):
    mesh = plsc.VectorSubcoreMesh(core_axis_name="c", subcore_axis_name="s")
    return pl.kernel(
        _sc_gather_body,
        out_type=(
            jax.ShapeDtypeStruct((_NW * nchunk, _CHUNK, emb), jnp.float32),
            jax.ShapeDtypeStruct((_NW * nchunk, _CHUNK, topics), jnp.float32),
        ),
        mesh=mesh,
        scratch_types=[
            pltpu.VMEM((nchunk, _CHUNK), jnp.int32),
            pltpu.VMEM((nchunk, _CHUNK), jnp.int32),
            pltpu.VMEM((nchunk, _CHUNK, emb), jnp.float32),
            pltpu.VMEM((nchunk, _CHUNK, topics), jnp.float32),
            pltpu.SemaphoreType.DMA,
        ],
        compiler_params=pltpu.CompilerParams(use_tc_tiling_on_sc=False),
    )


def _tc_combine_body(dw_ref, wv_ref, topics_ref, out_ref):
    dw = dw_ref[...]
    m = jnp.max(dw, axis=1, keepdims=True)
    e = jnp.exp(dw - m)
    p = e / jnp.sum(e, axis=1, keepdims=True)
    out_ref[...] = wv_ref[...] + jnp.dot(
        p, topics_ref[...], preferred_element_type=jnp.float32)


def _tc_combine(dw, wv, topic_embeds, blk):
    b, emb = wv.shape
    topics = dw.shape[1]
    grid = (b // blk,)
    return pl.pallas_call(
        _tc_combine_body,
        grid=grid,
        in_specs=[
            pl.BlockSpec((blk, topics), lambda i: (i, 0)),
            pl.BlockSpec((blk, emb), lambda i: (i, 0)),
            pl.BlockSpec((topics, emb), lambda i: (0, 0)),
        ],
        out_specs=pl.BlockSpec((blk, emb), lambda i: (i, 0)),
        out_shape=jax.ShapeDtypeStruct((b, emb), jnp.float32),
    )(dw, wv, topic_embeds)


def kernel(center_id, doc_id, word_embeds, doc_weights, topic_embeds):
    b = center_id.shape[0]
    emb = word_embeds.shape[1]
    topics = doc_weights.shape[1]
    nchunk = b // (_NW * _CHUNK)

    cid = center_id.reshape(_NW * nchunk, _CHUNK).astype(jnp.int32)
    did = doc_id.reshape(_NW * nchunk, _CHUNK).astype(jnp.int32)

    wv, dw = _make_sc_gather(b, emb, topics, nchunk)(
        cid, did, word_embeds, doc_weights)
    wv = wv.reshape(b, emb)
    dw = dw.reshape(b, topics)
    return _tc_combine(dw, wv, topic_embeds, blk=2048)


# speedup vs baseline: 1.6006x; 1.6006x over previous
import functools

import jax
import jax.numpy as jnp
from jax import lax
from jax.experimental import pallas as pl
from jax.experimental.pallas import tpu as pltpu
from jax.experimental.pallas import tpu_sc as plsc

_NC = 2
_NS = 16
_NW = _NC * _NS
_BURST = 64


def _sc_gather_body(cid_hbm, did_hbm, words_hbm, dweights_hbm,
                    wv_out, dw_out,
                    cidx_v, didx_v, wrows_v, drows_v, sem):
    n = cidx_v.shape[0]
    wid = lax.axis_index("s") * _NC + lax.axis_index("c")
    base = wid * n
    pltpu.sync_copy(cid_hbm.at[pl.ds(base, n)], cidx_v)
    pltpu.sync_copy(did_hbm.at[pl.ds(base, n)], didx_v)

    def burst(t, _):
        off = t * _BURST
        copies = []
        for g in range(_BURST // 16):
            cvec = cidx_v[pl.ds(off + g * 16, 16)]
            dvec = didx_v[pl.ds(off + g * 16, 16)]
            for k in range(16):
                j = g * 16 + k
                copies.append(pltpu.async_copy(
                    words_hbm.at[pl.ds(cvec[k], 1)],
                    wrows_v.at[pl.ds(j, 1)], sem))
                copies.append(pltpu.async_copy(
                    dweights_hbm.at[pl.ds(dvec[k], 1)],
                    drows_v.at[pl.ds(j, 1)], sem))
        for c in copies:
            c.wait()
        pltpu.sync_copy(wrows_v, wv_out.at[pl.ds(base + off, _BURST)])
        pltpu.sync_copy(drows_v, dw_out.at[pl.ds(base + off, _BURST)])
        return 0

    lax.fori_loop(0, n // _BURST, burst, 0)


def _make_sc_gather(b, emb, topics):
    n = b // _NW
    mesh = plsc.VectorSubcoreMesh(core_axis_name="c", subcore_axis_name="s")
    return pl.kernel(
        _sc_gather_body,
        out_type=(
            jax.ShapeDtypeStruct((b, emb), jnp.float32),
            jax.ShapeDtypeStruct((b, topics), jnp.float32),
        ),
        mesh=mesh,
        scratch_types=[
            pltpu.VMEM((n,), jnp.int32),
            pltpu.VMEM((n,), jnp.int32),
            pltpu.VMEM((_BURST, emb), jnp.float32),
            pltpu.VMEM((_BURST, topics), jnp.float32),
            pltpu.SemaphoreType.DMA,
        ],
    )


def _tc_combine_body(dw_ref, wv_ref, topics_ref, out_ref):
    dw = dw_ref[...]
    m = jnp.max(dw, axis=1, keepdims=True)
    e = jnp.exp(dw - m)
    p = e / jnp.sum(e, axis=1, keepdims=True)
    out_ref[...] = wv_ref[...] + jnp.dot(
        p, topics_ref[...], preferred_element_type=jnp.float32)


def _tc_combine(dw, wv, topic_embeds, blk):
    b, emb = wv.shape
    topics = dw.shape[1]
    grid = (b // blk,)
    return pl.pallas_call(
        _tc_combine_body,
        grid=grid,
        in_specs=[
            pl.BlockSpec((blk, topics), lambda i: (i, 0)),
            pl.BlockSpec((blk, emb), lambda i: (i, 0)),
            pl.BlockSpec((topics, emb), lambda i: (0, 0)),
        ],
        out_specs=pl.BlockSpec((blk, emb), lambda i: (i, 0)),
        out_shape=jax.ShapeDtypeStruct((b, emb), jnp.float32),
    )(dw, wv, topic_embeds)


def kernel(center_id, doc_id, word_embeds, doc_weights, topic_embeds):
    b = center_id.shape[0]
    emb = word_embeds.shape[1]
    topics = doc_weights.shape[1]

    cid = center_id.reshape(b).astype(jnp.int32)
    did = doc_id.reshape(b).astype(jnp.int32)

    wv, dw = _make_sc_gather(b, emb, topics)(
        cid, did, word_embeds, doc_weights)
    return _tc_combine(dw, wv, topic_embeds, blk=2048)


# speedup vs baseline: 2.5233x; 1.5765x over previous
import functools

import jax
import jax.numpy as jnp
from jax import lax
from jax.experimental import pallas as pl
from jax.experimental.pallas import tpu as pltpu
from jax.experimental.pallas import tpu_sc as plsc

_NC = 2
_NS = 16
_NW = _NC * _NS
_BURST = 128
_NSTAGE = 6


def _sc_gather_body(cid_hbm, did_hbm, wt_hbm, dweights_hbm,
                    wvT_out, dw_out,
                    cidx_v, didx_v, st0, st1, st2, st3, st4, st5,
                    wcolsT_v, drows_v, wsem, dsem):
    n = cidx_v.shape[0]
    emb = wt_hbm.shape[0]
    stages = [st0, st1, st2, st3, st4, st5]
    wid = lax.axis_index("s") * _NC + lax.axis_index("c")
    base = wid * n
    pltpu.sync_copy(cid_hbm.at[pl.ds(base, n)], cidx_v)
    pltpu.sync_copy(did_hbm.at[pl.ds(base, n)], didx_v)
    lanes = lax.iota(jnp.int32, 16)

    def burst(t, _):
        off = t * _BURST
        dcopies = []
        wcopies = [None] * 16
        for g in range(_BURST // 16):
            cvec = cidx_v[pl.ds(off + g * 16, 16)]
            dvec = didx_v[pl.ds(off + g * 16, 16)]
            for k in range(16):
                dcopies.append(pltpu.async_copy(
                    dweights_hbm.at[pl.ds(dvec[k], 1)],
                    drows_v.at[pl.ds(g * 16 + k, 1)], dsem))
            def issue(k):
                v0 = pl.multiple_of((cvec[k] >> 7) * 128, 128)
                return pltpu.async_copy(
                    wt_hbm.at[:, pl.ds(v0, 128)], stages[k % _NSTAGE], wsem)
            for k in range(_NSTAGE):
                wcopies[k] = issue(k)
            for k in range(16):
                wcopies[k].wait()
                ln = jnp.full((16,), cvec[k] & 127, jnp.int32)
                col_idx = jnp.full((16,), g * 16 + k, jnp.int32)
                for d0 in range(0, emb, 16):
                    col = plsc.load_gather(stages[k % _NSTAGE],
                                           [lanes + d0, ln])
                    plsc.store_scatter(wcolsT_v, [lanes + d0, col_idx], col)
                if k + _NSTAGE < 16:
                    wcopies[k + _NSTAGE] = issue(k + _NSTAGE)
        for c in dcopies:
            c.wait()
        pltpu.sync_copy(wcolsT_v, wvT_out.at[:, pl.ds(base + off, _BURST)])
        pltpu.sync_copy(drows_v, dw_out.at[pl.ds(base + off, _BURST)])
        return 0

    lax.fori_loop(0, n // _BURST, burst, 0)


def _make_sc_gather(b, emb, topics):
    n = b // _NW
    mesh = plsc.VectorSubcoreMesh(core_axis_name="c", subcore_axis_name="s")
    return pl.kernel(
        _sc_gather_body,
        out_type=(
            jax.ShapeDtypeStruct((emb, b), jnp.float32),
            jax.ShapeDtypeStruct((b, topics), jnp.float32),
        ),
        mesh=mesh,
        scratch_types=[
            pltpu.VMEM((n,), jnp.int32),
            pltpu.VMEM((n,), jnp.int32),
            pltpu.VMEM((emb, 128), jnp.float32),
            pltpu.VMEM((emb, 128), jnp.float32),
            pltpu.VMEM((emb, 128), jnp.float32),
            pltpu.VMEM((emb, 128), jnp.float32),
            pltpu.VMEM((emb, 128), jnp.float32),
            pltpu.VMEM((emb, 128), jnp.float32),
            pltpu.VMEM((emb, _BURST), jnp.float32),
            pltpu.VMEM((_BURST, topics), jnp.float32),
            pltpu.SemaphoreType.DMA,
            pltpu.SemaphoreType.DMA,
        ],
        compiler_params=pltpu.CompilerParams(needs_layout_passes=False),
    )


def _tc_combine_body(dw_ref, wvT_ref, topicsT_ref, out_ref):
    dw = dw_ref[...]
    m = jnp.max(dw, axis=1, keepdims=True)
    e = jnp.exp(dw - m)
    p = e / jnp.sum(e, axis=1, keepdims=True)
    doc = lax.dot_general(topicsT_ref[...], p, (((1,), (1,)), ((), ())),
                          preferred_element_type=jnp.float32)
    out_ref[...] = wvT_ref[...] + doc


def _tc_combine(dw, wvT, topicsT, blk):
    emb, b = wvT.shape
    topics = dw.shape[1]
    grid = (b // blk,)
    return pl.pallas_call(
        _tc_combine_body,
        grid=grid,
        in_specs=[
            pl.BlockSpec((blk, topics), lambda i: (i, 0)),
            pl.BlockSpec((emb, blk), lambda i: (0, i)),
            pl.BlockSpec((emb, topics), lambda i: (0, 0)),
        ],
        out_specs=pl.BlockSpec((emb, blk), lambda i: (0, i)),
        out_shape=jax.ShapeDtypeStruct((emb, b), jnp.float32),
    )(dw, wvT, topicsT)


def kernel(center_id, doc_id, word_embeds, doc_weights, topic_embeds):
    b = center_id.shape[0]
    emb = word_embeds.shape[1]
    topics = doc_weights.shape[1]

    cid = center_id.reshape(b).astype(jnp.int32)
    did = doc_id.reshape(b).astype(jnp.int32)

    wt = word_embeds.T
    topicsT = topic_embeds.T

    wvT, dw = _make_sc_gather(b, emb, topics)(cid, did, wt, doc_weights)
    outT = _tc_combine(dw, wvT, topicsT, blk=2048)
    return outT.T
